# baseline (speedup 1.0000x reference)
import functools

import jax
import jax.numpy as jnp
from jax import lax
from jax.experimental import pallas as pl
from jax.experimental.pallas import tpu as pltpu

_NUM_CLASSES = 80
_REG_MAX = 16
_NBINS = _REG_MAX + 1


def _loss_body(pd_ref, pb_ref, ap_ref, lab_ref, ab_ref, sc_ref, ssum_ref,
               l1_ref, iou_ref, dfl_ref, acc_ref):
    step = pl.program_id(0)
    nsteps = pl.num_programs(0)

    @pl.when(step == 0)
    def _init():
        for i in range(5):
            acc_ref[i] = 0.0

    lab = lab_ref[...]
    mask = lab != _NUM_CLASSES
    maskf = mask.astype(jnp.float32)

    pb = pb_ref[...]
    ab = ab_ref[...]
    l1sum = jnp.sum(jnp.abs(pb - ab) * maskf)

    x0p = pb[:, 0:1]
    x1p = pb[:, 1:2]
    x0a = ab[:, 0:1]
    x1a = ab[:, 1:2]
    inter = jnp.maximum(jnp.minimum(x1p, x1a) - jnp.maximum(x0p, x0a), 0.0)
    union = (x1p - x0p) + (x1a - x0a) - inter
    union_safe = jnp.where(mask, union, 1.0)
    tiou = jnp.where(mask, inter / union_safe, 0.0)
    iousum = jnp.sum(jnp.where(mask, 1.0 - tiou, 0.0))
    npos = jnp.sum(maskf)

    bw = jnp.sum(sc_ref[...], axis=1, keepdims=True) * maskf
    bwsum = jnp.sum(bw)

    ap = ap_ref[...]
    ltrb_l = jnp.clip(ap - x0a, 0.0, _REG_MAX - 0.01)
    ltrb_r = jnp.clip(x1a - ap, 0.0, _REG_MAX - 0.01)
    pd = pd_ref[...]
    iota = lax.broadcasted_iota(jnp.int32, pd[:, :_NBINS].shape, 1)

    def _dfl_half(x, ltrb):
        logS = jnp.log(jnp.sum(jnp.exp(x), axis=1, keepdims=True))
        t = ltrb.astype(jnp.int32)
        xt = jnp.sum(jnp.where(iota == t, x, 0.0), axis=1, keepdims=True)
        xt1 = jnp.sum(jnp.where(iota == t + 1, x, 0.0), axis=1, keepdims=True)
        wl = (t + 1).astype(jnp.float32) - ltrb
        wr = 1.0 - wl
        return (logS - xt) * wl + (logS - xt1) * wr

    dfl = 0.5 * (_dfl_half(pd[:, :_NBINS], ltrb_l)
                 + _dfl_half(pd[:, _NBINS:], ltrb_r))
    dflsum = jnp.sum(dfl * bw)

    acc_ref[0] += npos
    acc_ref[1] += l1sum
    acc_ref[2] += iousum
    acc_ref[3] += bwsum
    acc_ref[4] += dflsum

    @pl.when(step == nsteps - 1)
    def _finish():
        np_ = acc_ref[0]
        ssum = ssum_ref[0]
        l1_ref[0] = acc_ref[1] / (np_ * 2.0)
        iou_ref[0] = (acc_ref[2] / np_) * acc_ref[3] / ssum
        dfl_ref[0] = acc_ref[4] / ssum


@functools.partial(jax.jit, static_argnames=("interpret",))
def _run(pred_dist, pred_bboxes, anchor_points, assigned_labels,
         assigned_bboxes, assigned_scores, assigned_scores_sum,
         interpret=False):
    B, L = assigned_labels.shape
    N = B * L
    C = 4096
    grid = (N // C,)
    pd = pred_dist.reshape(N, 2 * _NBINS)
    pb = pred_bboxes.reshape(N, 2)
    ap = anchor_points.reshape(N, 1)
    lab = assigned_labels.reshape(N, 1)
    ab = assigned_bboxes.reshape(N, 2)
    sc = assigned_scores.reshape(N, _NUM_CLASSES)
    ssum = assigned_scores_sum.reshape(1)

    out = pl.pallas_call(
        _loss_body,
        grid=grid,
        in_specs=[
            pl.BlockSpec((C, 2 * _NBINS), lambda i: (i, 0)),
            pl.BlockSpec((C, 2), lambda i: (i, 0)),
            pl.BlockSpec((C, 1), lambda i: (i, 0)),
            pl.BlockSpec((C, 1), lambda i: (i, 0)),
            pl.BlockSpec((C, 2), lambda i: (i, 0)),
            pl.BlockSpec((C, _NUM_CLASSES), lambda i: (i, 0)),
            pl.BlockSpec(memory_space=pltpu.SMEM),
        ],
        out_specs=[
            pl.BlockSpec(memory_space=pltpu.SMEM),
            pl.BlockSpec(memory_space=pltpu.SMEM),
            pl.BlockSpec(memory_space=pltpu.SMEM),
        ],
        out_shape=[
            jax.ShapeDtypeStruct((1,), jnp.float32),
            jax.ShapeDtypeStruct((1,), jnp.float32),
            jax.ShapeDtypeStruct((1,), jnp.float32),
        ],
        scratch_shapes=[pltpu.SMEM((8,), jnp.float32)],
        compiler_params=pltpu.CompilerParams(
            dimension_semantics=("arbitrary",)),
        interpret=interpret,
    )(pd, pb, ap, lab, ab, sc, ssum)
    return (out[0][0], out[1][0], out[2][0])


def kernel(pred_dist, pred_bboxes, anchor_points, assigned_labels,
           assigned_bboxes, assigned_scores, assigned_scores_sum):
    return _run(pred_dist, pred_bboxes, anchor_points, assigned_labels,
                assigned_bboxes, assigned_scores, assigned_scores_sum)


# speedup vs baseline: 3.8837x; 3.8837x over previous
import functools

import jax
import jax.numpy as jnp
from jax import lax
from jax.experimental import pallas as pl
from jax.experimental.pallas import tpu as pltpu

_NUM_CLASSES = 80
_REG_MAX = 16
_NBINS = _REG_MAX + 1
_LANES = 128


def _loss_body(pd_ref, sc_ref, x0p_ref, x1p_ref, ap_ref, lab_ref, x0a_ref,
               x1a_ref, ssum_ref, l1_ref, iou_ref, dfl_ref, acc_ref):
    step = pl.program_id(0)
    nsteps = pl.num_programs(0)

    @pl.when(step == 0)
    def _init():
        for i in range(5):
            acc_ref[i] = 0.0

    mask = lab_ref[...] != _NUM_CLASSES
    maskf = mask.astype(jnp.float32)

    x0p = x0p_ref[...]
    x1p = x1p_ref[...]
    x0a = x0a_ref[...]
    x1a = x1a_ref[...]
    l1sum = jnp.sum((jnp.abs(x0p - x0a) + jnp.abs(x1p - x1a)) * maskf)

    inter = jnp.maximum(jnp.minimum(x1p, x1a) - jnp.maximum(x0p, x0a), 0.0)
    union = (x1p - x0p) + (x1a - x0a) - inter
    union_safe = jnp.where(mask, union, 1.0)
    tiou = jnp.where(mask, inter / union_safe, 0.0)
    iousum = jnp.sum(jnp.where(mask, 1.0 - tiou, 0.0))
    npos = jnp.sum(maskf)

    bw = jnp.sum(sc_ref[...], axis=0) * maskf
    bwsum = jnp.sum(bw)

    ap = ap_ref[...]
    ltrb_l = jnp.clip(ap - x0a, 0.0, _REG_MAX - 0.01)
    ltrb_r = jnp.clip(x1a - ap, 0.0, _REG_MAX - 0.01)
    pd = pd_ref[...]
    iota3 = lax.broadcasted_iota(jnp.int32, (_NBINS,) + ap.shape, 0)

    def _dfl_half(x, ltrb):
        logS = jnp.log(jnp.sum(jnp.exp(x), axis=0))
        t = ltrb.astype(jnp.int32)
        xt = jnp.sum(jnp.where(iota3 == t[None], x, 0.0), axis=0)
        xt1 = jnp.sum(jnp.where(iota3 == t[None] + 1, x, 0.0), axis=0)
        wl = (t + 1).astype(jnp.float32) - ltrb
        wr = 1.0 - wl
        return (logS - xt) * wl + (logS - xt1) * wr

    dfl = 0.5 * (_dfl_half(pd[:_NBINS], ltrb_l) + _dfl_half(pd[_NBINS:], ltrb_r))
    dflsum = jnp.sum(dfl * bw)

    acc_ref[0] += npos
    acc_ref[1] += l1sum
    acc_ref[2] += iousum
    acc_ref[3] += bwsum
    acc_ref[4] += dflsum

    @pl.when(step == nsteps - 1)
    def _finish():
        np_ = acc_ref[0]
        ssum = ssum_ref[0]
        l1_ref[0] = acc_ref[1] / (np_ * 2.0)
        iou_ref[0] = (acc_ref[2] / np_) * acc_ref[3] / ssum
        dfl_ref[0] = acc_ref[4] / ssum


@functools.partial(jax.jit, static_argnames=("interpret",))
def _run(pred_dist, pred_bboxes, anchor_points, assigned_labels,
         assigned_bboxes, assigned_scores, assigned_scores_sum,
         interpret=False):
    B, L = assigned_labels.shape
    N = B * L
    NR = N // _LANES
    RB = 64
    grid = (NR // RB,)

    pdT = pred_dist.reshape(N, 2 * _NBINS).T.reshape(2 * _NBINS, NR, _LANES)
    scT = assigned_scores.reshape(N, _NUM_CLASSES).T.reshape(
        _NUM_CLASSES, NR, _LANES)
    x0p = pred_bboxes[..., 0].reshape(NR, _LANES)
    x1p = pred_bboxes[..., 1].reshape(NR, _LANES)
    ap = anchor_points.reshape(NR, _LANES)
    lab = assigned_labels.reshape(NR, _LANES)
    x0a = assigned_bboxes[..., 0].reshape(NR, _LANES)
    x1a = assigned_bboxes[..., 1].reshape(NR, _LANES)
    ssum = assigned_scores_sum.reshape(1)

    row_spec = pl.BlockSpec((RB, _LANES), lambda i: (i, 0))
    out = pl.pallas_call(
        _loss_body,
        grid=grid,
        in_specs=[
            pl.BlockSpec((2 * _NBINS, RB, _LANES), lambda i: (0, i, 0)),
            pl.BlockSpec((_NUM_CLASSES, RB, _LANES), lambda i: (0, i, 0)),
            row_spec, row_spec, row_spec, row_spec, row_spec, row_spec,
            pl.BlockSpec(memory_space=pltpu.SMEM),
        ],
        out_specs=[
            pl.BlockSpec(memory_space=pltpu.SMEM),
            pl.BlockSpec(memory_space=pltpu.SMEM),
            pl.BlockSpec(memory_space=pltpu.SMEM),
        ],
        out_shape=[
            jax.ShapeDtypeStruct((1,), jnp.float32),
            jax.ShapeDtypeStruct((1,), jnp.float32),
            jax.ShapeDtypeStruct((1,), jnp.float32),
        ],
        scratch_shapes=[pltpu.SMEM((8,), jnp.float32)],
        compiler_params=pltpu.CompilerParams(
            dimension_semantics=("arbitrary",)),
        interpret=interpret,
    )(pdT, scT, x0p, x1p, ap, lab, x0a, x1a, ssum)
    return (out[0][0], out[1][0], out[2][0])


def kernel(pred_dist, pred_bboxes, anchor_points, assigned_labels,
           assigned_bboxes, assigned_scores, assigned_scores_sum):
    return _run(pred_dist, pred_bboxes, anchor_points, assigned_labels,
                assigned_bboxes, assigned_scores, assigned_scores_sum)
